# baseline (speedup 1.0000x reference)
import functools

import jax
import jax.numpy as jnp
from jax.experimental import pallas as pl
from jax.experimental.pallas import tpu as pltpu

M_POINTS = 1024
GRASP_THR = 0.1
_NB = 2048


def _head_body(f_ref, w_ref, b_ref, masked_ref, grasp_ref):
    f = f_ref[0]
    w = w_ref[...]
    h = jax.lax.dot_general(
        w, f, (((1,), (0,)), ((), ())), preferred_element_type=jnp.float32
    )
    h = h + b_ref[...]
    obj = h[1:2] > h[0:1]
    g = h[2:3]
    keep = obj & (g > GRASP_THR)
    masked_ref[0] = jnp.where(keep, g, jnp.float32(-1e9))
    grasp_ref[0] = g


def kernel(point_clouds, features, W, b):
    B, C, N = features.shape
    nblk = pl.cdiv(N, _NB)
    masked, grasp = pl.pallas_call(
        _head_body,
        grid=(B, nblk),
        in_specs=[
            pl.BlockSpec((1, C, _NB), lambda bi, ni: (bi, 0, ni)),
            pl.BlockSpec((3, C), lambda bi, ni: (0, 0)),
            pl.BlockSpec((3, 1), lambda bi, ni: (0, 0)),
        ],
        out_specs=[
            pl.BlockSpec((1, 1, _NB), lambda bi, ni: (bi, 0, ni)),
            pl.BlockSpec((1, 1, _NB), lambda bi, ni: (bi, 0, ni)),
        ],
        out_shape=[
            jax.ShapeDtypeStruct((B, 1, N), jnp.float32),
            jax.ShapeDtypeStruct((B, 1, N), jnp.float32),
        ],
    )(features, W, b.reshape(3, 1))
    masked = masked.reshape(B, N)
    grasp = grasp.reshape(B, N)

    _, idx = jax.lax.top_k(masked, M_POINTS)

    xyz = jnp.take_along_axis(point_clouds, idx[:, :, None], axis=1)
    feats = jnp.take_along_axis(
        jnp.transpose(features, (0, 2, 1)), idx[:, :, None], axis=1
    )
    scores = jnp.take_along_axis(grasp, idx, axis=1)[:, :, None]
    return jnp.concatenate([xyz, feats, scores], axis=-1)
